# Initial kernel scaffold; baseline (speedup 1.0000x reference)
import jax
import jax.numpy as jnp
from jax.experimental import pallas as pl


def kernel(feature, adj, g1_W, g1_a, g1_Wout, g1_aout, g2_W, g2_a, g2_Wout, g2_aout, att_W1, att_b1, att_W2):
    raise NotImplementedError("write your pallas kernel here")


# speedup vs baseline: 1.2589x; 1.2589x over previous
import functools

import jax
import jax.numpy as jnp
from jax.experimental import pallas as pl

ALPHA = 0.2
NEG = -9e15
RB = 256


def _leaky(x):
    return jnp.where(x >= 0, x, ALPHA * x)


def _elu(x):
    return jnp.where(x > 0, x, jnp.exp(jnp.minimum(x, 0.0)) - 1.0)


def _masked_softmax_matmul(mask, e, wh):
    att = jnp.where(mask, _leaky(e), NEG)
    mx = jnp.max(att, axis=1, keepdims=True)
    p = jnp.exp(att - mx)
    s = jnp.sum(p, axis=1, keepdims=True)
    num = jnp.dot(p, wh, preferred_element_type=jnp.float32)
    return num / s


def _proj_kernel(feat_ref, w_ref, alo_ref, ahi_ref, wh_ref, e1_ref, e2_ref):
    x = feat_ref[...]
    for k in range(w_ref.shape[0]):
        wh = jnp.dot(x, w_ref[k], preferred_element_type=jnp.float32)
        wh_ref[k] = wh
        e1_ref[:, k:k + 1] = jnp.dot(wh, alo_ref[k], preferred_element_type=jnp.float32)
        e2_ref[:, k:k + 1] = jnp.dot(wh, ahi_ref[k], preferred_element_type=jnp.float32)


def _heads_kernel(adj0_ref, adj1_ref, wh_ref, e1_ref, e2t_ref,
                  h1_ref, h2_ref, m0_ref, m1_ref, *, fhid):
    for b, (adj_ref, m_ref, h_ref) in enumerate(
            ((adj0_ref, m0_ref, h1_ref), (adj1_ref, m1_ref, h2_ref))):
        mask = adj_ref[...] > 0
        m_ref[...] = mask.astype(jnp.int8)
        for h in range(2):
            k = 2 * b + h
            e = e1_ref[:, k:k + 1] + e2t_ref[k:k + 1, :]
            hp = _masked_softmax_matmul(mask, e, wh_ref[k])
            h_ref[:, h * fhid:(h + 1) * fhid] = _elu(hp)


def _out_proj_kernel(h1_ref, h2_ref, wout_ref, alo_ref, ahi_ref,
                     who_ref, eo1_ref, eo2_ref):
    for b, h_ref in enumerate((h1_ref, h2_ref)):
        who = jnp.dot(h_ref[...], wout_ref[b], preferred_element_type=jnp.float32)
        who_ref[b] = who
        eo1_ref[:, b:b + 1] = jnp.dot(who, alo_ref[b], preferred_element_type=jnp.float32)
        eo2_ref[:, b:b + 1] = jnp.dot(who, ahi_ref[b], preferred_element_type=jnp.float32)


def _out_kernel(m0_ref, m1_ref, who_ref, eo1_ref, eo2t_ref,
                aw1_ref, ab1_ref, aw2_ref, out_ref):
    embs, ws = [], []
    for b, m_ref in enumerate((m0_ref, m1_ref)):
        mask = m_ref[...].astype(jnp.int32) > 0
        e = eo1_ref[:, b:b + 1] + eo2t_ref[b:b + 1, :]
        emb = _elu(_masked_softmax_matmul(mask, e, who_ref[b]))
        w = jnp.dot(jnp.tanh(jnp.dot(emb, aw1_ref[...],
                                     preferred_element_type=jnp.float32) + ab1_ref[...]),
                    aw2_ref[...], preferred_element_type=jnp.float32)
        embs.append(emb)
        ws.append(w)
    wmax = jnp.maximum(ws[0], ws[1])
    p0 = jnp.exp(ws[0] - wmax)
    p1 = jnp.exp(ws[1] - wmax)
    out_ref[...] = (p0 * embs[0] + p1 * embs[1]) / (p0 + p1)


def kernel(feature, adj, g1_W, g1_a, g1_Wout, g1_aout,
           g2_W, g2_a, g2_Wout, g2_aout, att_W1, att_b1, att_W2):
    n, fin = feature.shape
    h_heads, _, fhid = g1_W.shape
    fout = g1_Wout.shape[1]
    nk = 2 * h_heads
    f32 = jnp.float32

    w_all = jnp.concatenate([g1_W, g2_W], axis=0)
    a_all = jnp.concatenate([g1_a, g2_a], axis=0)
    a_lo, a_hi = a_all[:, :fhid, :], a_all[:, fhid:, :]

    wh, e1, e2 = pl.pallas_call(
        _proj_kernel,
        out_shape=(
            jax.ShapeDtypeStruct((nk, n, fhid), f32),
            jax.ShapeDtypeStruct((n, nk), f32),
            jax.ShapeDtypeStruct((n, nk), f32),
        ),
    )(feature, w_all, a_lo, a_hi)
    e2t = e2.T

    grid = (n // RB,)
    h1, h2, m0, m1 = pl.pallas_call(
        functools.partial(_heads_kernel, fhid=fhid),
        grid=grid,
        in_specs=[
            pl.BlockSpec((RB, n), lambda i: (i, 0)),
            pl.BlockSpec((RB, n), lambda i: (i, 0)),
            pl.BlockSpec((nk, n, fhid), lambda i: (0, 0, 0)),
            pl.BlockSpec((RB, nk), lambda i: (i, 0)),
            pl.BlockSpec((nk, n), lambda i: (0, 0)),
        ],
        out_specs=[
            pl.BlockSpec((RB, 2 * fhid), lambda i: (i, 0)),
            pl.BlockSpec((RB, 2 * fhid), lambda i: (i, 0)),
            pl.BlockSpec((RB, n), lambda i: (i, 0)),
            pl.BlockSpec((RB, n), lambda i: (i, 0)),
        ],
        out_shape=(
            jax.ShapeDtypeStruct((n, 2 * fhid), f32),
            jax.ShapeDtypeStruct((n, 2 * fhid), f32),
            jax.ShapeDtypeStruct((n, n), jnp.int8),
            jax.ShapeDtypeStruct((n, n), jnp.int8),
        ),
    )(adj[0], adj[1], wh, e1, e2t)

    wout_all = jnp.stack([g1_Wout, g2_Wout], axis=0)
    aout_all = jnp.stack([g1_aout, g2_aout], axis=0)
    ao_lo, ao_hi = aout_all[:, :fout, :], aout_all[:, fout:, :]
    who, eo1, eo2 = pl.pallas_call(
        _out_proj_kernel,
        out_shape=(
            jax.ShapeDtypeStruct((2, n, fout), f32),
            jax.ShapeDtypeStruct((n, 2), f32),
            jax.ShapeDtypeStruct((n, 2), f32),
        ),
    )(h1, h2, wout_all, ao_lo, ao_hi)
    eo2t = eo2.T

    out = pl.pallas_call(
        _out_kernel,
        grid=grid,
        in_specs=[
            pl.BlockSpec((RB, n), lambda i: (i, 0)),
            pl.BlockSpec((RB, n), lambda i: (i, 0)),
            pl.BlockSpec((2, n, fout), lambda i: (0, 0, 0)),
            pl.BlockSpec((RB, 2), lambda i: (i, 0)),
            pl.BlockSpec((2, n), lambda i: (0, 0)),
            pl.BlockSpec((fout, att_W1.shape[1]), lambda i: (0, 0)),
            pl.BlockSpec((1, att_b1.shape[0]), lambda i: (0, 0)),
            pl.BlockSpec((att_W2.shape[0], 1), lambda i: (0, 0)),
        ],
        out_specs=pl.BlockSpec((RB, fout), lambda i: (i, 0)),
        out_shape=jax.ShapeDtypeStruct((n, fout), f32),
    )(m0, m1, who, eo1, eo2t, att_W1, att_b1.reshape(1, -1), att_W2)
    return out


# speedup vs baseline: 2.6788x; 2.1279x over previous
import functools

import jax
import jax.numpy as jnp
from jax.experimental import pallas as pl

ALPHA = 0.2
RB = 256


def _elu(x):
    return jnp.where(x > 0, x, jnp.exp(jnp.minimum(x, 0.0)) - 1.0)


def _att_pool(maskf, e1col, e2row, e2m, wh_ext, fdim):
    m_i = e1col + e2m
    m_i = jnp.maximum(m_i, ALPHA * m_i)
    t1 = e1col - m_i
    t2 = ALPHA * e1col - m_i
    u = jnp.maximum(t1 + e2row, t2 + ALPHA * e2row)
    p = jnp.exp(u) * maskf
    ne = jnp.dot(p, wh_ext, preferred_element_type=jnp.float32)
    return ne[:, :fdim] / (ne[:, fdim:fdim + 1] + 1e-30)


def _proj_kernel(feat_ref, w_ref, alo_ref, ahi_ref, wh_ref, e1_ref, e2_ref, e2m_ref):
    x = feat_ref[...]
    fhid = w_ref.shape[2]
    for k in range(w_ref.shape[0]):
        wh = jnp.dot(x, w_ref[k], preferred_element_type=jnp.float32)
        wh_ref[k, :, :fhid] = wh
        wh_ref[k, :, fhid:] = jnp.ones((x.shape[0], 1), jnp.float32)
        e1_ref[:, k:k + 1] = jnp.dot(wh, alo_ref[k], preferred_element_type=jnp.float32)
        e2 = jnp.dot(wh, ahi_ref[k], preferred_element_type=jnp.float32)
        e2_ref[:, k:k + 1] = e2
        e2m_ref[0:1, k:k + 1] = jnp.max(e2, axis=0, keepdims=True)


def _heads_kernel(adj_ref, wh_ref, e1_ref, e2t_ref, e2m_ref,
                  h1_ref, h2_ref, m0_ref, m1_ref, *, fhid):
    for b, (m_ref, h_ref) in enumerate(((m0_ref, h1_ref), (m1_ref, h2_ref))):
        adji = adj_ref[b]
        m_ref[...] = adji.astype(jnp.int8)
        maskf = adji.astype(jnp.float32)
        for h in range(2):
            k = 2 * b + h
            hp = _att_pool(maskf, e1_ref[:, k:k + 1], e2t_ref[k:k + 1, :],
                           e2m_ref[0:1, k:k + 1], wh_ref[k], fhid)
            h_ref[:, h * fhid:(h + 1) * fhid] = _elu(hp)


def _out_proj_kernel(h1_ref, h2_ref, wout_ref, alo_ref, ahi_ref,
                     who_ref, eo1_ref, eo2_ref, eo2m_ref):
    fout = wout_ref.shape[2]
    for b, h_ref in enumerate((h1_ref, h2_ref)):
        who = jnp.dot(h_ref[...], wout_ref[b], preferred_element_type=jnp.float32)
        who_ref[b, :, :fout] = who
        who_ref[b, :, fout:] = jnp.ones((who.shape[0], 1), jnp.float32)
        eo1_ref[:, b:b + 1] = jnp.dot(who, alo_ref[b], preferred_element_type=jnp.float32)
        eo2 = jnp.dot(who, ahi_ref[b], preferred_element_type=jnp.float32)
        eo2_ref[:, b:b + 1] = eo2
        eo2m_ref[0:1, b:b + 1] = jnp.max(eo2, axis=0, keepdims=True)


def _out_kernel(m0_ref, m1_ref, who_ref, eo1_ref, eo2t_ref, eo2m_ref,
                aw1_ref, ab1_ref, aw2_ref, out_ref, *, fout):
    embs, ws = [], []
    for b, m_ref in enumerate((m0_ref, m1_ref)):
        maskf = m_ref[...].astype(jnp.float32)
        hp = _att_pool(maskf, eo1_ref[:, b:b + 1], eo2t_ref[b:b + 1, :],
                       eo2m_ref[0:1, b:b + 1], who_ref[b], fout)
        emb = _elu(hp)
        w = jnp.dot(jnp.tanh(jnp.dot(emb, aw1_ref[...],
                                     preferred_element_type=jnp.float32) + ab1_ref[...]),
                    aw2_ref[...], preferred_element_type=jnp.float32)
        embs.append(emb)
        ws.append(w)
    wmax = jnp.maximum(ws[0], ws[1])
    p0 = jnp.exp(ws[0] - wmax)
    p1 = jnp.exp(ws[1] - wmax)
    out_ref[...] = (p0 * embs[0] + p1 * embs[1]) / (p0 + p1)


def kernel(feature, adj, g1_W, g1_a, g1_Wout, g1_aout,
           g2_W, g2_a, g2_Wout, g2_aout, att_W1, att_b1, att_W2):
    n, fin = feature.shape
    h_heads, _, fhid = g1_W.shape
    fout = g1_Wout.shape[1]
    nk = 2 * h_heads
    f32 = jnp.float32

    w_all = jnp.concatenate([g1_W, g2_W], axis=0)
    a_all = jnp.concatenate([g1_a, g2_a], axis=0)
    a_lo, a_hi = a_all[:, :fhid, :], a_all[:, fhid:, :]

    wh, e1, e2, e2m = pl.pallas_call(
        _proj_kernel,
        out_shape=(
            jax.ShapeDtypeStruct((nk, n, fhid + 1), f32),
            jax.ShapeDtypeStruct((n, nk), f32),
            jax.ShapeDtypeStruct((n, nk), f32),
            jax.ShapeDtypeStruct((1, nk), f32),
        ),
    )(feature, w_all, a_lo, a_hi)
    e2t = e2.T

    grid = (n // RB,)
    h1, h2, m0, m1 = pl.pallas_call(
        functools.partial(_heads_kernel, fhid=fhid),
        grid=grid,
        in_specs=[
            pl.BlockSpec((2, RB, n), lambda i: (0, i, 0)),
            pl.BlockSpec((nk, n, fhid + 1), lambda i: (0, 0, 0)),
            pl.BlockSpec((RB, nk), lambda i: (i, 0)),
            pl.BlockSpec((nk, n), lambda i: (0, 0)),
            pl.BlockSpec((1, nk), lambda i: (0, 0)),
        ],
        out_specs=[
            pl.BlockSpec((RB, 2 * fhid), lambda i: (i, 0)),
            pl.BlockSpec((RB, 2 * fhid), lambda i: (i, 0)),
            pl.BlockSpec((RB, n), lambda i: (i, 0)),
            pl.BlockSpec((RB, n), lambda i: (i, 0)),
        ],
        out_shape=(
            jax.ShapeDtypeStruct((n, 2 * fhid), f32),
            jax.ShapeDtypeStruct((n, 2 * fhid), f32),
            jax.ShapeDtypeStruct((n, n), jnp.int8),
            jax.ShapeDtypeStruct((n, n), jnp.int8),
        ),
    )(adj, wh, e1, e2t, e2m)

    wout_all = jnp.stack([g1_Wout, g2_Wout], axis=0)
    aout_all = jnp.stack([g1_aout, g2_aout], axis=0)
    ao_lo, ao_hi = aout_all[:, :fout, :], aout_all[:, fout:, :]
    who, eo1, eo2, eo2m = pl.pallas_call(
        _out_proj_kernel,
        out_shape=(
            jax.ShapeDtypeStruct((2, n, fout + 1), f32),
            jax.ShapeDtypeStruct((n, 2), f32),
            jax.ShapeDtypeStruct((n, 2), f32),
            jax.ShapeDtypeStruct((1, 2), f32),
        ),
    )(h1, h2, wout_all, ao_lo, ao_hi)
    eo2t = eo2.T

    out = pl.pallas_call(
        functools.partial(_out_kernel, fout=fout),
        grid=grid,
        in_specs=[
            pl.BlockSpec((RB, n), lambda i: (i, 0)),
            pl.BlockSpec((RB, n), lambda i: (i, 0)),
            pl.BlockSpec((2, n, fout + 1), lambda i: (0, 0, 0)),
            pl.BlockSpec((RB, 2), lambda i: (i, 0)),
            pl.BlockSpec((2, n), lambda i: (0, 0)),
            pl.BlockSpec((1, 2), lambda i: (0, 0)),
            pl.BlockSpec((fout, att_W1.shape[1]), lambda i: (0, 0)),
            pl.BlockSpec((1, att_b1.shape[0]), lambda i: (0, 0)),
            pl.BlockSpec((att_W2.shape[0], 1), lambda i: (0, 0)),
        ],
        out_specs=pl.BlockSpec((RB, fout), lambda i: (i, 0)),
        out_shape=jax.ShapeDtypeStruct((n, fout), f32),
    )(m0, m1, who, eo1, eo2t, eo2m, att_W1, att_b1.reshape(1, -1), att_W2)
    return out


# speedup vs baseline: 3.0561x; 1.1409x over previous
import functools

import jax
import jax.numpy as jnp
from jax.experimental import pallas as pl
from jax.experimental.pallas import tpu as pltpu
from jax.scipy.linalg import block_diag as _block_diag

ALPHA = 0.2
LOG2E = 1.4426950408889634
RBA = 256
RBB = 512


def _elu(x):
    return jnp.where(x > 0, x, jnp.exp(jnp.minimum(x, 0.0)) - 1.0)


def _att_pool(maskf, e1col, e2row, e2m, wh_ext, fdim):
    m_i = e1col + e2m
    m_i = jnp.maximum(m_i, ALPHA * m_i)
    t1 = (e1col - m_i) * LOG2E
    t2 = (ALPHA * e1col - m_i) * LOG2E
    e2s = e2row * LOG2E
    e2as = e2row * (ALPHA * LOG2E)
    u = jnp.maximum(t1 + e2s, t2 + e2as)
    p = jnp.exp2(u) * maskf
    ne = jnp.dot(p, wh_ext, preferred_element_type=jnp.float32)
    return ne[:, :fdim] / (ne[:, fdim:fdim + 1] + 1e-30)


def _stage_a(adj_ref, x_ref, xt_ref, wcat_ref, wcatt_ref, alobd_ref, ahibd_ref,
             h1_ref, h2_ref, m0_ref, m1_ref,
             whext_s, e1_s, e2t_s, e2m_s, *, fhid, nk, rb):
    i = pl.program_id(0)

    @pl.when(i == 0)
    def _proj():
        wh = jnp.dot(x_ref[...], wcat_ref[...], preferred_element_type=jnp.float32)
        wht = jnp.dot(wcatt_ref[...], xt_ref[...], preferred_element_type=jnp.float32)
        e1_s[...] = jnp.dot(wh, alobd_ref[...], preferred_element_type=jnp.float32)
        e2t = jnp.dot(ahibd_ref[...], wht, preferred_element_type=jnp.float32)
        e2t_s[...] = e2t
        e2m_s[...] = jnp.max(e2t, axis=1, keepdims=True)
        ones = jnp.ones((wh.shape[0], 1), jnp.float32)
        for k in range(nk):
            whext_s[k, :, :fhid] = wh[:, k * fhid:(k + 1) * fhid]
            whext_s[k, :, fhid:] = ones

    for b, (m_ref, h_ref) in enumerate(((m0_ref, h1_ref), (m1_ref, h2_ref))):
        adji = adj_ref[b]
        m_ref[...] = adji.astype(jnp.int8)
        maskf = adji.astype(jnp.float32)
        for h in range(2):
            k = 2 * b + h
            hp = _att_pool(maskf, e1_s[pl.ds(i * rb, rb), k:k + 1],
                           e2t_s[k:k + 1, :], e2m_s[k:k + 1, 0:1],
                           whext_s[k], fhid)
            h_ref[:, h * fhid:(h + 1) * fhid] = _elu(hp)


def _stage_b(m0_ref, m1_ref, h1_ref, h2_ref, ht1_ref, ht2_ref,
             wout_ref, alo_ref, vat_ref, aw1_ref, ab1_ref, aw2_ref,
             out_ref, whoext_s, eo1_s, eo2t_s, eo2m_s, *, fout, rb):
    i = pl.program_id(0)

    @pl.when(i == 0)
    def _proj():
        for b, (h_ref, ht_ref) in enumerate(((h1_ref, ht1_ref), (h2_ref, ht2_ref))):
            who = jnp.dot(h_ref[...], wout_ref[b], preferred_element_type=jnp.float32)
            whoext_s[b, :, :fout] = who
            whoext_s[b, :, fout:] = jnp.ones((who.shape[0], 1), jnp.float32)
            eo1_s[:, b:b + 1] = jnp.dot(who, alo_ref[b], preferred_element_type=jnp.float32)
            eo2t = jnp.dot(vat_ref[b], ht_ref[...], preferred_element_type=jnp.float32)
            eo2t_s[b:b + 1, :] = eo2t
            eo2m_s[b:b + 1, 0:1] = jnp.max(eo2t, axis=1, keepdims=True)

    embs, ws = [], []
    for b, m_ref in enumerate((m0_ref, m1_ref)):
        maskf = m_ref[...].astype(jnp.float32)
        hp = _att_pool(maskf, eo1_s[pl.ds(i * rb, rb), b:b + 1],
                       eo2t_s[b:b + 1, :], eo2m_s[b:b + 1, 0:1],
                       whoext_s[b], fout)
        emb = _elu(hp)
        w = jnp.dot(jnp.tanh(jnp.dot(emb, aw1_ref[...],
                                     preferred_element_type=jnp.float32) + ab1_ref[...]),
                    aw2_ref[...], preferred_element_type=jnp.float32)
        embs.append(emb)
        ws.append(w)
    wmax = jnp.maximum(ws[0], ws[1])
    p0 = jnp.exp(ws[0] - wmax)
    p1 = jnp.exp(ws[1] - wmax)
    out_ref[...] = (p0 * embs[0] + p1 * embs[1]) / (p0 + p1)


def kernel(feature, adj, g1_W, g1_a, g1_Wout, g1_aout,
           g2_W, g2_a, g2_Wout, g2_aout, att_W1, att_b1, att_W2):
    n, fin = feature.shape
    h_heads, _, fhid = g1_W.shape
    fout = g1_Wout.shape[1]
    nk = 2 * h_heads
    f32 = jnp.float32

    w_all = jnp.concatenate([g1_W, g2_W], axis=0)
    a_all = jnp.concatenate([g1_a, g2_a], axis=0)
    a_lo, a_hi = a_all[:, :fhid, :], a_all[:, fhid:, :]
    wcat = jnp.transpose(w_all, (1, 0, 2)).reshape(fin, nk * fhid)
    wcatt = wcat.T
    alobd = _block_diag(*[a_lo[k] for k in range(nk)])
    ahibd = _block_diag(*[a_hi[k].T for k in range(nk)])
    xt = feature.T

    h1, h2, m0, m1 = pl.pallas_call(
        functools.partial(_stage_a, fhid=fhid, nk=nk, rb=RBA),
        grid=(n // RBA,),
        in_specs=[
            pl.BlockSpec((2, RBA, n), lambda i: (0, i, 0)),
            pl.BlockSpec((n, fin), lambda i: (0, 0)),
            pl.BlockSpec((fin, n), lambda i: (0, 0)),
            pl.BlockSpec((fin, nk * fhid), lambda i: (0, 0)),
            pl.BlockSpec((nk * fhid, fin), lambda i: (0, 0)),
            pl.BlockSpec((nk * fhid, nk), lambda i: (0, 0)),
            pl.BlockSpec((nk, nk * fhid), lambda i: (0, 0)),
        ],
        out_specs=[
            pl.BlockSpec((RBA, 2 * fhid), lambda i: (i, 0)),
            pl.BlockSpec((RBA, 2 * fhid), lambda i: (i, 0)),
            pl.BlockSpec((RBA, n), lambda i: (i, 0)),
            pl.BlockSpec((RBA, n), lambda i: (i, 0)),
        ],
        out_shape=(
            jax.ShapeDtypeStruct((n, 2 * fhid), f32),
            jax.ShapeDtypeStruct((n, 2 * fhid), f32),
            jax.ShapeDtypeStruct((n, n), jnp.int8),
            jax.ShapeDtypeStruct((n, n), jnp.int8),
        ),
        scratch_shapes=[
            pltpu.VMEM((nk, n, fhid + 1), f32),
            pltpu.VMEM((n, nk), f32),
            pltpu.VMEM((nk, n), f32),
            pltpu.VMEM((nk, 1), f32),
        ],
    )(adj, feature, xt, wcat, wcatt, alobd, ahibd)

    wout_all = jnp.stack([g1_Wout, g2_Wout], axis=0)
    aout_all = jnp.stack([g1_aout, g2_aout], axis=0)
    ao_lo, ao_hi = aout_all[:, :fout, :], aout_all[:, fout:, :]
    vat = jnp.swapaxes(wout_all @ ao_hi, 1, 2)
    ht1, ht2 = h1.T, h2.T

    out = pl.pallas_call(
        functools.partial(_stage_b, fout=fout, rb=RBB),
        grid=(n // RBB,),
        in_specs=[
            pl.BlockSpec((RBB, n), lambda i: (i, 0)),
            pl.BlockSpec((RBB, n), lambda i: (i, 0)),
            pl.BlockSpec((n, 2 * fhid), lambda i: (0, 0)),
            pl.BlockSpec((n, 2 * fhid), lambda i: (0, 0)),
            pl.BlockSpec((2 * fhid, n), lambda i: (0, 0)),
            pl.BlockSpec((2 * fhid, n), lambda i: (0, 0)),
            pl.BlockSpec((2, 2 * fhid, fout), lambda i: (0, 0, 0)),
            pl.BlockSpec((2, fout, 1), lambda i: (0, 0, 0)),
            pl.BlockSpec((2, 1, 2 * fhid), lambda i: (0, 0, 0)),
            pl.BlockSpec((fout, att_W1.shape[1]), lambda i: (0, 0)),
            pl.BlockSpec((1, att_b1.shape[0]), lambda i: (0, 0)),
            pl.BlockSpec((att_W2.shape[0], 1), lambda i: (0, 0)),
        ],
        out_specs=pl.BlockSpec((RBB, fout), lambda i: (i, 0)),
        out_shape=jax.ShapeDtypeStruct((n, fout), f32),
        scratch_shapes=[
            pltpu.VMEM((2, n, fout + 1), f32),
            pltpu.VMEM((n, 2), f32),
            pltpu.VMEM((2, n), f32),
            pltpu.VMEM((2, 1), f32),
        ],
    )(m0, m1, h1, h2, ht1, ht2, wout_all, ao_lo, vat,
      att_W1, att_b1.reshape(1, -1), att_W2)
    return out


# speedup vs baseline: 3.1545x; 1.0322x over previous
import functools

import jax
import jax.numpy as jnp
from jax.experimental import pallas as pl
from jax.experimental.pallas import tpu as pltpu
from jax.scipy.linalg import block_diag as _block_diag

ALPHA = 0.2
LOG2E = 1.4426950408889634
RBA = 512
RBB = 1024


def _elu(x):
    return jnp.where(x > 0, x, jnp.exp(jnp.minimum(x, 0.0)) - 1.0)


def _att_pool(maskf, e1col, e2row, e2m, wh_ext, fdim):
    m_i = e1col + e2m
    m_i = jnp.maximum(m_i, ALPHA * m_i)
    t1 = (e1col - m_i) * LOG2E
    t2 = (ALPHA * e1col - m_i) * LOG2E
    e2s = e2row * LOG2E
    e2as = e2row * (ALPHA * LOG2E)
    u = jnp.maximum(t1 + e2s, t2 + e2as)
    p = jnp.exp2(u) * maskf
    ne = jnp.dot(p, wh_ext, preferred_element_type=jnp.float32)
    return ne[:, :fdim] / (ne[:, fdim:fdim + 1] + 1e-30)


def _stage_a(adj_ref, x_ref, xt_ref, wcat_ref, wcatt_ref, alobd_ref, ahibd_ref,
             h_ref, m_ref, whext_s, e1_s, e2t_s, e2m_s, *, fhid, nk, rb):
    b = pl.program_id(0)
    i = pl.program_id(1)

    @pl.when(jnp.logical_and(b == 0, i == 0))
    def _proj():
        wh = jnp.dot(x_ref[...], wcat_ref[...], preferred_element_type=jnp.float32)
        wht = jnp.dot(wcatt_ref[...], xt_ref[...], preferred_element_type=jnp.float32)
        e1mat = jnp.dot(wh, alobd_ref[...], preferred_element_type=jnp.float32)
        e2t = jnp.dot(ahibd_ref[...], wht, preferred_element_type=jnp.float32)
        e2t_s[...] = e2t
        e2m_s[...] = jnp.max(e2t, axis=1, keepdims=True)
        ones = jnp.ones((wh.shape[0], 1), jnp.float32)
        for k in range(nk):
            whext_s[k, :, :fhid] = wh[:, k * fhid:(k + 1) * fhid]
            whext_s[k, :, fhid:] = ones
        e1_s[...] = e1mat

    adji = adj_ref[0]
    m_ref[0] = adji.astype(jnp.int8)
    maskf = adji.astype(jnp.float32)
    b0 = b == 0
    e1blk = e1_s[pl.ds(i * rb, rb), :]
    for h in range(2):
        e1col = jnp.where(b0, e1blk[:, h:h + 1], e1blk[:, 2 + h:3 + h])
        e2row = jnp.where(b0, e2t_s[h:h + 1, :], e2t_s[2 + h:3 + h, :])
        e2m = jnp.where(b0, e2m_s[h:h + 1, :], e2m_s[2 + h:3 + h, :])
        wh_ext = whext_s[pl.ds(2 * b + h, 1)].reshape(e2t_s.shape[1], fhid + 1)
        hp = _att_pool(maskf, e1col, e2row, e2m, wh_ext, fhid)
        h_ref[0, :, h * fhid:(h + 1) * fhid] = _elu(hp)


def _stage_b(m_ref, h_ref, ht_ref, wout_ref, alo_ref, vat_ref,
             aw1_ref, ab1_ref, aw2_ref,
             out_ref, whoext_s, eo1_s, eo2t_s, eo2m_s, *, fout, rb):
    i = pl.program_id(0)

    @pl.when(i == 0)
    def _proj():
        for b in range(2):
            who = jnp.dot(h_ref[b], wout_ref[b], preferred_element_type=jnp.float32)
            whoext_s[b, :, :fout] = who
            whoext_s[b, :, fout:] = jnp.ones((who.shape[0], 1), jnp.float32)
            eo1_s[b] = jnp.dot(who, alo_ref[b], preferred_element_type=jnp.float32)
            eo2t = jnp.dot(vat_ref[b], ht_ref[b], preferred_element_type=jnp.float32)
            eo2t_s[b:b + 1, :] = eo2t
            eo2m_s[b:b + 1, 0:1] = jnp.max(eo2t, axis=1, keepdims=True)

    embs, ws = [], []
    for b in range(2):
        maskf = m_ref[b].astype(jnp.float32)
        e1col = eo1_s[b, pl.ds(i * rb, rb), :]
        hp = _att_pool(maskf, e1col, eo2t_s[b:b + 1, :], eo2m_s[b:b + 1, 0:1],
                       whoext_s[b], fout)
        emb = _elu(hp)
        w = jnp.dot(jnp.tanh(jnp.dot(emb, aw1_ref[...],
                                     preferred_element_type=jnp.float32) + ab1_ref[...]),
                    aw2_ref[...], preferred_element_type=jnp.float32)
        embs.append(emb)
        ws.append(w)
    wmax = jnp.maximum(ws[0], ws[1])
    p0 = jnp.exp(ws[0] - wmax)
    p1 = jnp.exp(ws[1] - wmax)
    out_ref[...] = (p0 * embs[0] + p1 * embs[1]) / (p0 + p1)


def kernel(feature, adj, g1_W, g1_a, g1_Wout, g1_aout,
           g2_W, g2_a, g2_Wout, g2_aout, att_W1, att_b1, att_W2):
    n, fin = feature.shape
    h_heads, _, fhid = g1_W.shape
    fout = g1_Wout.shape[1]
    nk = 2 * h_heads
    f32 = jnp.float32

    w_all = jnp.concatenate([g1_W, g2_W], axis=0)
    a_all = jnp.concatenate([g1_a, g2_a], axis=0)
    a_lo, a_hi = a_all[:, :fhid, :], a_all[:, fhid:, :]
    wcat = jnp.transpose(w_all, (1, 0, 2)).reshape(fin, nk * fhid)
    wcatt = wcat.T
    alobd = _block_diag(*[a_lo[k] for k in range(nk)])
    ahibd = _block_diag(*[a_hi[k].T for k in range(nk)])
    xt = feature.T

    h_all, m_all = pl.pallas_call(
        functools.partial(_stage_a, fhid=fhid, nk=nk, rb=RBA),
        grid=(2, n // RBA),
        in_specs=[
            pl.BlockSpec((1, RBA, n), lambda b, i: (b, i, 0)),
            pl.BlockSpec((n, fin), lambda b, i: (0, 0)),
            pl.BlockSpec((fin, n), lambda b, i: (0, 0)),
            pl.BlockSpec((fin, nk * fhid), lambda b, i: (0, 0)),
            pl.BlockSpec((nk * fhid, fin), lambda b, i: (0, 0)),
            pl.BlockSpec((nk * fhid, nk), lambda b, i: (0, 0)),
            pl.BlockSpec((nk, nk * fhid), lambda b, i: (0, 0)),
        ],
        out_specs=[
            pl.BlockSpec((1, RBA, 2 * fhid), lambda b, i: (b, i, 0)),
            pl.BlockSpec((1, RBA, n), lambda b, i: (b, i, 0)),
        ],
        out_shape=(
            jax.ShapeDtypeStruct((2, n, 2 * fhid), f32),
            jax.ShapeDtypeStruct((2, n, n), jnp.int8),
        ),
        scratch_shapes=[
            pltpu.VMEM((nk, n, fhid + 1), f32),
            pltpu.VMEM((n, nk), f32),
            pltpu.VMEM((nk, n), f32),
            pltpu.VMEM((nk, 1), f32),
        ],
    )(adj, feature, xt, wcat, wcatt, alobd, ahibd)

    wout_all = jnp.stack([g1_Wout, g2_Wout], axis=0)
    aout_all = jnp.stack([g1_aout, g2_aout], axis=0)
    ao_lo, ao_hi = aout_all[:, :fout, :], aout_all[:, fout:, :]
    vat = jnp.swapaxes(wout_all @ ao_hi, 1, 2)
    ht_all = jnp.swapaxes(h_all, 1, 2)

    out = pl.pallas_call(
        functools.partial(_stage_b, fout=fout, rb=RBB),
        grid=(n // RBB,),
        in_specs=[
            pl.BlockSpec((2, RBB, n), lambda i: (0, i, 0)),
            pl.BlockSpec((2, n, 2 * fhid), lambda i: (0, 0, 0)),
            pl.BlockSpec((2, 2 * fhid, n), lambda i: (0, 0, 0)),
            pl.BlockSpec((2, 2 * fhid, fout), lambda i: (0, 0, 0)),
            pl.BlockSpec((2, fout, 1), lambda i: (0, 0, 0)),
            pl.BlockSpec((2, 1, 2 * fhid), lambda i: (0, 0, 0)),
            pl.BlockSpec((fout, att_W1.shape[1]), lambda i: (0, 0)),
            pl.BlockSpec((1, att_b1.shape[0]), lambda i: (0, 0)),
            pl.BlockSpec((att_W2.shape[0], 1), lambda i: (0, 0)),
        ],
        out_specs=pl.BlockSpec((RBB, fout), lambda i: (i, 0)),
        out_shape=jax.ShapeDtypeStruct((n, fout), f32),
        scratch_shapes=[
            pltpu.VMEM((2, n, fout + 1), f32),
            pltpu.VMEM((2, n, 1), f32),
            pltpu.VMEM((2, n), f32),
            pltpu.VMEM((2, 1), f32),
        ],
    )(m_all, h_all, ht_all, wout_all, ao_lo, vat,
      att_W1, att_b1.reshape(1, -1), att_W2)
    return out


# speedup vs baseline: 3.3853x; 1.0732x over previous
import functools

import jax
import jax.numpy as jnp
from jax.experimental import pallas as pl
from jax.experimental.pallas import tpu as pltpu
from jax.scipy.linalg import block_diag as _block_diag

ALPHA = 0.2
LOG2E = 1.4426950408889634
RBA = 512
RBB = 512


def _elu(x):
    return jnp.where(x > 0, x, jnp.exp(jnp.minimum(x, 0.0)) - 1.0)


def _att_pool(maskb, e1col, e2row, e2m, wh_ext, fdim):
    bf = jnp.bfloat16
    m_i = e1col + e2m
    m_i = jnp.maximum(m_i, ALPHA * m_i)
    t1 = ((e1col - m_i) * LOG2E).astype(bf)
    t2 = ((ALPHA * e1col - m_i) * LOG2E).astype(bf)
    e2s = (e2row * LOG2E).astype(bf)
    e2as = (e2row * (ALPHA * LOG2E)).astype(bf)
    u = jnp.maximum(t1 + e2s, t2 + e2as)
    p = jnp.exp2(u) * maskb
    ne = jnp.dot(p, wh_ext, preferred_element_type=jnp.float32)
    return ne[:, :fdim] / (ne[:, fdim:fdim + 1] + 1e-30)


def _stage_a(adj_ref, x_ref, xt_ref, wcat_ref, wcatt_ref, alobd_ref, ahibd_ref,
             h_ref, m_ref, whext_s, e1_s, e2t_s, e2m_s, *, fhid, nk, rb):
    b = pl.program_id(0)
    i = pl.program_id(1)

    @pl.when(jnp.logical_and(b == 0, i == 0))
    def _proj():
        wh = jnp.dot(x_ref[...], wcat_ref[...], preferred_element_type=jnp.float32)
        wht = jnp.dot(wcatt_ref[...], xt_ref[...], preferred_element_type=jnp.float32)
        e1mat = jnp.dot(wh, alobd_ref[...], preferred_element_type=jnp.float32)
        e2t = jnp.dot(ahibd_ref[...], wht, preferred_element_type=jnp.float32)
        e2t_s[...] = e2t
        e2m_s[...] = jnp.max(e2t, axis=1, keepdims=True)
        ones = jnp.ones((wh.shape[0], 1), jnp.bfloat16)
        for k in range(nk):
            whext_s[k, :, :fhid] = wh[:, k * fhid:(k + 1) * fhid].astype(jnp.bfloat16)
            whext_s[k, :, fhid:] = ones
        e1_s[...] = e1mat

    adji = adj_ref[0]
    m_ref[0] = adji.astype(jnp.int8)
    maskb = adji.astype(jnp.bfloat16)
    b0 = b == 0
    e1blk = e1_s[pl.ds(i * rb, rb), :]
    for h in range(2):
        e1col = jnp.where(b0, e1blk[:, h:h + 1], e1blk[:, 2 + h:3 + h])
        e2row = jnp.where(b0, e2t_s[h:h + 1, :], e2t_s[2 + h:3 + h, :])
        e2m = jnp.where(b0, e2m_s[h:h + 1, :], e2m_s[2 + h:3 + h, :])
        wh_ext = whext_s[pl.ds(2 * b + h, 1)].reshape(e2t_s.shape[1], fhid + 1)
        hp = _att_pool(maskb, e1col, e2row, e2m, wh_ext, fhid)
        h_ref[0, :, h * fhid:(h + 1) * fhid] = _elu(hp)


def _stage_b(m_ref, h_ref, ht_ref, wout_ref, alo_ref, vat_ref,
             aw1_ref, ab1_ref, aw2_ref,
             out_ref, whoext_s, eo1_s, eo2t_s, eo2m_s, *, fout, rb):
    i = pl.program_id(0)

    @pl.when(i == 0)
    def _proj():
        for b in range(2):
            who = jnp.dot(h_ref[b], wout_ref[b], preferred_element_type=jnp.float32)
            whoext_s[b, :, :fout] = who.astype(jnp.bfloat16)
            whoext_s[b, :, fout:] = jnp.ones((who.shape[0], 1), jnp.bfloat16)
            eo1_s[b] = jnp.dot(who, alo_ref[b], preferred_element_type=jnp.float32)
            eo2t = jnp.dot(vat_ref[b], ht_ref[b], preferred_element_type=jnp.float32)
            eo2t_s[b:b + 1, :] = eo2t
            eo2m_s[b:b + 1, 0:1] = jnp.max(eo2t, axis=1, keepdims=True)

    embs, ws = [], []
    for b in range(2):
        maskb = m_ref[b].astype(jnp.bfloat16)
        e1col = eo1_s[b, pl.ds(i * rb, rb), :]
        hp = _att_pool(maskb, e1col, eo2t_s[b:b + 1, :], eo2m_s[b:b + 1, 0:1],
                       whoext_s[b], fout)
        emb = _elu(hp)
        w = jnp.dot(jnp.tanh(jnp.dot(emb, aw1_ref[...],
                                     preferred_element_type=jnp.float32) + ab1_ref[...]),
                    aw2_ref[...], preferred_element_type=jnp.float32)
        embs.append(emb)
        ws.append(w)
    wmax = jnp.maximum(ws[0], ws[1])
    p0 = jnp.exp(ws[0] - wmax)
    p1 = jnp.exp(ws[1] - wmax)
    out_ref[...] = (p0 * embs[0] + p1 * embs[1]) / (p0 + p1)


def kernel(feature, adj, g1_W, g1_a, g1_Wout, g1_aout,
           g2_W, g2_a, g2_Wout, g2_aout, att_W1, att_b1, att_W2):
    n, fin = feature.shape
    h_heads, _, fhid = g1_W.shape
    fout = g1_Wout.shape[1]
    nk = 2 * h_heads
    f32 = jnp.float32

    w_all = jnp.concatenate([g1_W, g2_W], axis=0)
    a_all = jnp.concatenate([g1_a, g2_a], axis=0)
    a_lo, a_hi = a_all[:, :fhid, :], a_all[:, fhid:, :]
    wcat = jnp.transpose(w_all, (1, 0, 2)).reshape(fin, nk * fhid)
    wcatt = wcat.T
    alobd = _block_diag(*[a_lo[k] for k in range(nk)])
    ahibd = _block_diag(*[a_hi[k].T for k in range(nk)])
    xt = feature.T

    h_all, m_all = pl.pallas_call(
        functools.partial(_stage_a, fhid=fhid, nk=nk, rb=RBA),
        grid=(2, n // RBA),
        in_specs=[
            pl.BlockSpec((1, RBA, n), lambda b, i: (b, i, 0)),
            pl.BlockSpec((n, fin), lambda b, i: (0, 0)),
            pl.BlockSpec((fin, n), lambda b, i: (0, 0)),
            pl.BlockSpec((fin, nk * fhid), lambda b, i: (0, 0)),
            pl.BlockSpec((nk * fhid, fin), lambda b, i: (0, 0)),
            pl.BlockSpec((nk * fhid, nk), lambda b, i: (0, 0)),
            pl.BlockSpec((nk, nk * fhid), lambda b, i: (0, 0)),
        ],
        out_specs=[
            pl.BlockSpec((1, RBA, 2 * fhid), lambda b, i: (b, i, 0)),
            pl.BlockSpec((1, RBA, n), lambda b, i: (b, i, 0)),
        ],
        out_shape=(
            jax.ShapeDtypeStruct((2, n, 2 * fhid), f32),
            jax.ShapeDtypeStruct((2, n, n), jnp.int8),
        ),
        scratch_shapes=[
            pltpu.VMEM((nk, n, fhid + 1), jnp.bfloat16),
            pltpu.VMEM((n, nk), f32),
            pltpu.VMEM((nk, n), f32),
            pltpu.VMEM((nk, 1), f32),
        ],
    )(adj, feature, xt, wcat, wcatt, alobd, ahibd)

    wout_all = jnp.stack([g1_Wout, g2_Wout], axis=0)
    aout_all = jnp.stack([g1_aout, g2_aout], axis=0)
    ao_lo, ao_hi = aout_all[:, :fout, :], aout_all[:, fout:, :]
    vat = jnp.swapaxes(wout_all @ ao_hi, 1, 2)
    ht_all = jnp.swapaxes(h_all, 1, 2)

    out = pl.pallas_call(
        functools.partial(_stage_b, fout=fout, rb=RBB),
        grid=(n // RBB,),
        in_specs=[
            pl.BlockSpec((2, RBB, n), lambda i: (0, i, 0)),
            pl.BlockSpec((2, n, 2 * fhid), lambda i: (0, 0, 0)),
            pl.BlockSpec((2, 2 * fhid, n), lambda i: (0, 0, 0)),
            pl.BlockSpec((2, 2 * fhid, fout), lambda i: (0, 0, 0)),
            pl.BlockSpec((2, fout, 1), lambda i: (0, 0, 0)),
            pl.BlockSpec((2, 1, 2 * fhid), lambda i: (0, 0, 0)),
            pl.BlockSpec((fout, att_W1.shape[1]), lambda i: (0, 0)),
            pl.BlockSpec((1, att_b1.shape[0]), lambda i: (0, 0)),
            pl.BlockSpec((att_W2.shape[0], 1), lambda i: (0, 0)),
        ],
        out_specs=pl.BlockSpec((RBB, fout), lambda i: (i, 0)),
        out_shape=jax.ShapeDtypeStruct((n, fout), f32),
        scratch_shapes=[
            pltpu.VMEM((2, n, fout + 1), jnp.bfloat16),
            pltpu.VMEM((2, n, 1), f32),
            pltpu.VMEM((2, n), f32),
            pltpu.VMEM((2, 1), f32),
        ],
    )(m_all, h_all, ht_all, wout_all, ao_lo, vat,
      att_W1, att_b1.reshape(1, -1), att_W2)
    return out


# speedup vs baseline: 3.4284x; 1.0127x over previous
import functools

import jax
import jax.numpy as jnp
from jax.experimental import pallas as pl
from jax.experimental.pallas import tpu as pltpu
from jax.scipy.linalg import block_diag as _block_diag

ALPHA = 0.2
LOG2E = 1.4426950408889634
RBA = 512
RBB = 1024


def _elu(x):
    return jnp.where(x > 0, x, jnp.exp(jnp.minimum(x, 0.0)) - 1.0)


def _att_pool(maskb, e1col, e2row, e2m, wh_ext, fdim):
    bf = jnp.bfloat16
    m_i = e1col + e2m
    m_i = jnp.maximum(m_i, ALPHA * m_i)
    t1 = ((e1col - m_i) * LOG2E).astype(bf)
    t2 = ((ALPHA * e1col - m_i) * LOG2E).astype(bf)
    e2s = (e2row * LOG2E).astype(bf)
    e2as = (e2row * (ALPHA * LOG2E)).astype(bf)
    u = jnp.maximum(t1 + e2s, t2 + e2as)
    p = jnp.exp2(u) * maskb
    ne = jnp.dot(p, wh_ext, preferred_element_type=jnp.float32)
    return ne[:, :fdim] / (ne[:, fdim:fdim + 1] + 1e-30)


def _stage_a(adj_ref, x_ref, xt_ref, wcat_ref, wcatt_ref, alobd_ref, ahibd_ref,
             h_ref, m_ref, whext_s, e1_s, e2t_s, e2m_s, *, fhid, nk, rb):
    b = pl.program_id(0)
    i = pl.program_id(1)

    @pl.when(jnp.logical_and(b == 0, i == 0))
    def _proj():
        wh = jnp.dot(x_ref[...], wcat_ref[...], preferred_element_type=jnp.float32)
        wht = jnp.dot(wcatt_ref[...], xt_ref[...], preferred_element_type=jnp.float32)
        e1mat = jnp.dot(wh, alobd_ref[...], preferred_element_type=jnp.float32)
        e2t = jnp.dot(ahibd_ref[...], wht, preferred_element_type=jnp.float32)
        e2t_s[...] = e2t
        e2m_s[...] = jnp.max(e2t, axis=1, keepdims=True)
        ones = jnp.ones((wh.shape[0], 1), jnp.bfloat16)
        for k in range(nk):
            whext_s[k, :, :fhid] = wh[:, k * fhid:(k + 1) * fhid].astype(jnp.bfloat16)
            whext_s[k, :, fhid:] = ones
        e1_s[...] = e1mat

    adji = adj_ref[0]
    m_ref[0] = adji.astype(jnp.int8)
    maskb = adji.astype(jnp.bfloat16)
    b0 = b == 0
    e1blk = e1_s[pl.ds(i * rb, rb), :]
    for h in range(2):
        e1col = jnp.where(b0, e1blk[:, h:h + 1], e1blk[:, 2 + h:3 + h])
        e2row = jnp.where(b0, e2t_s[h:h + 1, :], e2t_s[2 + h:3 + h, :])
        e2m = jnp.where(b0, e2m_s[h:h + 1, :], e2m_s[2 + h:3 + h, :])
        wh_ext = whext_s[pl.ds(2 * b + h, 1)].reshape(e2t_s.shape[1], fhid + 1)
        hp = _att_pool(maskb, e1col, e2row, e2m, wh_ext, fhid)
        h_ref[0, :, h * fhid:(h + 1) * fhid] = _elu(hp)


def _stage_b(m_ref, h_ref, ht_ref, wout_ref, alo_ref, vat_ref,
             aw1_ref, ab1_ref, aw2_ref,
             out_ref, whoext_s, eo1_s, eo2t_s, eo2m_s, *, fout, rb):
    i = pl.program_id(0)

    @pl.when(i == 0)
    def _proj():
        for b in range(2):
            who = jnp.dot(h_ref[b], wout_ref[b], preferred_element_type=jnp.float32)
            whoext_s[b, :, :fout] = who.astype(jnp.bfloat16)
            whoext_s[b, :, fout:] = jnp.ones((who.shape[0], 1), jnp.bfloat16)
            eo1_s[b] = jnp.dot(who, alo_ref[b], preferred_element_type=jnp.float32)
            eo2t = jnp.dot(vat_ref[b], ht_ref[b], preferred_element_type=jnp.float32)
            eo2t_s[b:b + 1, :] = eo2t
            eo2m_s[b:b + 1, 0:1] = jnp.max(eo2t, axis=1, keepdims=True)

    embs, ws = [], []
    for b in range(2):
        maskb = m_ref[b].astype(jnp.bfloat16)
        e1col = eo1_s[b, pl.ds(i * rb, rb), :]
        hp = _att_pool(maskb, e1col, eo2t_s[b:b + 1, :], eo2m_s[b:b + 1, 0:1],
                       whoext_s[b], fout)
        emb = _elu(hp)
        w = jnp.dot(jnp.tanh(jnp.dot(emb, aw1_ref[...],
                                     preferred_element_type=jnp.float32) + ab1_ref[...]),
                    aw2_ref[...], preferred_element_type=jnp.float32)
        embs.append(emb)
        ws.append(w)
    wmax = jnp.maximum(ws[0], ws[1])
    p0 = jnp.exp(ws[0] - wmax)
    p1 = jnp.exp(ws[1] - wmax)
    out_ref[...] = (p0 * embs[0] + p1 * embs[1]) / (p0 + p1)


def kernel(feature, adj, g1_W, g1_a, g1_Wout, g1_aout,
           g2_W, g2_a, g2_Wout, g2_aout, att_W1, att_b1, att_W2):
    n, fin = feature.shape
    h_heads, _, fhid = g1_W.shape
    fout = g1_Wout.shape[1]
    nk = 2 * h_heads
    f32 = jnp.float32

    w_all = jnp.concatenate([g1_W, g2_W], axis=0)
    a_all = jnp.concatenate([g1_a, g2_a], axis=0)
    a_lo, a_hi = a_all[:, :fhid, :], a_all[:, fhid:, :]
    wcat = jnp.transpose(w_all, (1, 0, 2)).reshape(fin, nk * fhid)
    wcatt = wcat.T
    alobd = _block_diag(*[a_lo[k] for k in range(nk)])
    ahibd = _block_diag(*[a_hi[k].T for k in range(nk)])
    xt = feature.T

    h_all, m_all = pl.pallas_call(
        functools.partial(_stage_a, fhid=fhid, nk=nk, rb=RBA),
        grid=(2, n // RBA),
        in_specs=[
            pl.BlockSpec((1, RBA, n), lambda b, i: (b, i, 0)),
            pl.BlockSpec((n, fin), lambda b, i: (0, 0)),
            pl.BlockSpec((fin, n), lambda b, i: (0, 0)),
            pl.BlockSpec((fin, nk * fhid), lambda b, i: (0, 0)),
            pl.BlockSpec((nk * fhid, fin), lambda b, i: (0, 0)),
            pl.BlockSpec((nk * fhid, nk), lambda b, i: (0, 0)),
            pl.BlockSpec((nk, nk * fhid), lambda b, i: (0, 0)),
        ],
        out_specs=[
            pl.BlockSpec((1, RBA, 2 * fhid), lambda b, i: (b, i, 0)),
            pl.BlockSpec((1, RBA, n), lambda b, i: (b, i, 0)),
        ],
        out_shape=(
            jax.ShapeDtypeStruct((2, n, 2 * fhid), f32),
            jax.ShapeDtypeStruct((2, n, n), jnp.int8),
        ),
        scratch_shapes=[
            pltpu.VMEM((nk, n, fhid + 1), jnp.bfloat16),
            pltpu.VMEM((n, nk), f32),
            pltpu.VMEM((nk, n), f32),
            pltpu.VMEM((nk, 1), f32),
        ],
    )(adj, feature, xt, wcat, wcatt, alobd, ahibd)

    wout_all = jnp.stack([g1_Wout, g2_Wout], axis=0)
    aout_all = jnp.stack([g1_aout, g2_aout], axis=0)
    ao_lo, ao_hi = aout_all[:, :fout, :], aout_all[:, fout:, :]
    vat = jnp.swapaxes(wout_all @ ao_hi, 1, 2)
    ht_all = jnp.swapaxes(h_all, 1, 2)

    out = pl.pallas_call(
        functools.partial(_stage_b, fout=fout, rb=RBB),
        grid=(n // RBB,),
        in_specs=[
            pl.BlockSpec((2, RBB, n), lambda i: (0, i, 0)),
            pl.BlockSpec((2, n, 2 * fhid), lambda i: (0, 0, 0)),
            pl.BlockSpec((2, 2 * fhid, n), lambda i: (0, 0, 0)),
            pl.BlockSpec((2, 2 * fhid, fout), lambda i: (0, 0, 0)),
            pl.BlockSpec((2, fout, 1), lambda i: (0, 0, 0)),
            pl.BlockSpec((2, 1, 2 * fhid), lambda i: (0, 0, 0)),
            pl.BlockSpec((fout, att_W1.shape[1]), lambda i: (0, 0)),
            pl.BlockSpec((1, att_b1.shape[0]), lambda i: (0, 0)),
            pl.BlockSpec((att_W2.shape[0], 1), lambda i: (0, 0)),
        ],
        out_specs=pl.BlockSpec((RBB, fout), lambda i: (i, 0)),
        out_shape=jax.ShapeDtypeStruct((n, fout), f32),
        scratch_shapes=[
            pltpu.VMEM((2, n, fout + 1), jnp.bfloat16),
            pltpu.VMEM((2, n, 1), f32),
            pltpu.VMEM((2, n), f32),
            pltpu.VMEM((2, 1), f32),
        ],
    )(m_all, h_all, ht_all, wout_all, ao_lo, vat,
      att_W1, att_b1.reshape(1, -1), att_W2)
    return out
